# Initial kernel scaffold; baseline (speedup 1.0000x reference)
import jax
import jax.numpy as jnp
from jax.experimental import pallas as pl


def kernel(x, tables):
    raise NotImplementedError("write your pallas kernel here")


# speedup vs baseline: 1.3609x; 1.3609x over previous
import functools

import numpy as np
import jax
import jax.numpy as jnp
from jax import lax
from jax.experimental import pallas as pl
from jax.experimental.pallas import tpu as pltpu
from jax.experimental.pallas import tpu_sc as plsc

NUM_LEVELS = 16
MIN_RES = 128
MAX_RES = 2048
LOG2_HASHMAP_SIZE = 19
HSIZE = 1 << LOG2_HASHMAP_SIZE
F = 2
_b = np.exp((np.log(MAX_RES) - np.log(MIN_RES)) / (NUM_LEVELS - 1))
_RES = [int(np.floor(MIN_RES * _b ** lvl)) for lvl in range(NUM_LEVELS)]

P1 = 73856093
P2 = 19349663
P3 = 83492791

NC = 2
NS = 16
L = 16
NW = NC * NS


@functools.lru_cache(maxsize=None)
def _build(N):
    NPW = N // NW
    C = 1024
    NCHUNK = NPW // C
    G = C // L
    W = 16 * C

    mesh = plsc.VectorSubcoreMesh(core_axis_name="c", subcore_axis_name="s")

    @functools.partial(
        pl.kernel,
        mesh=mesh,
        out_type=jax.ShapeDtypeStruct((NUM_LEVELS * F, N), jnp.float32),
        scratch_types=[
            pltpu.VMEM((C,), jnp.float32),
            pltpu.VMEM((C,), jnp.float32),
            pltpu.VMEM((C,), jnp.float32),
            pltpu.VMEM((C,), jnp.float32),
            pltpu.VMEM((C,), jnp.float32),
            pltpu.VMEM((C,), jnp.float32),
            pltpu.VMEM((W,), jnp.int32),
            pltpu.VMEM((W,), jnp.float32),
            pltpu.VMEM((C * NUM_LEVELS * F,), jnp.float32),
            pltpu.SemaphoreType.DMA,
        ],
    )
    def hash_enc(xs, ys, zs, tbl, out, xb, yb, zb, wxb, wyb, wzb, idxb,
                 rowsb, outb, sem):
        wid = lax.axis_index("s") * NC + lax.axis_index("c")
        lane = lax.iota(jnp.int32, L)

        def chunk_body(ci, carry):
            pbase = wid * jnp.int32(NPW) + ci * jnp.int32(C)
            pltpu.sync_copy(xs.at[pl.ds(pbase, C)], xb)
            pltpu.sync_copy(ys.at[pl.ds(pbase, C)], yb)
            pltpu.sync_copy(zs.at[pl.ds(pbase, C)], zb)

            for lvl in range(NUM_LEVELS):
                res = float(_RES[lvl])
                resm1 = jnp.int32(_RES[lvl] - 1)
                lvbase2 = jnp.int32(lvl << (LOG2_HASHMAP_SIZE + 1))
                mask = jnp.int32(HSIZE - 1)

                def idx_body(g, c2, lvbase2=lvbase2, resm1=resm1, res=res,
                             mask=mask):
                    p = g * jnp.int32(L)
                    xv = xb[pl.ds(p, L)]
                    yv = yb[pl.ds(p, L)]
                    zv = zb[pl.ds(p, L)]
                    sx = xv * res
                    sy = yv * res
                    sz = zv * res
                    tx = sx.astype(jnp.int32)
                    ty = sy.astype(jnp.int32)
                    tz = sz.astype(jnp.int32)
                    wxb[pl.ds(p, L)] = sx - tx.astype(jnp.float32)
                    wyb[pl.ds(p, L)] = sy - ty.astype(jnp.float32)
                    wzb[pl.ds(p, L)] = sz - tz.astype(jnp.float32)
                    ix = jnp.minimum(tx, resm1)
                    iy = jnp.minimum(ty, resm1)
                    iz = jnp.minimum(tz, resm1)
                    hx0 = ix * jnp.int32(P1)
                    hx1 = hx0 + jnp.int32(P1)
                    hy0 = iy * jnp.int32(P2)
                    hy1 = hy0 + jnp.int32(P2)
                    hz0 = iz * jnp.int32(P3)
                    hz1 = hz0 + jnp.int32(P3)
                    p2 = p + p
                    c = 0
                    for hx in (hx0, hx1):
                        hxy0 = hx ^ hy0
                        hxy1 = hx ^ hy1
                        for hxy in (hxy0, hxy1):
                            for hz in (hz0, hz1):
                                h = hxy ^ hz
                                v0 = ((h & mask) << 1) | lvbase2
                                base2 = jnp.int32(c * 2 * C) + p2
                                idxb[pl.ds(base2, L)] = v0
                                idxb[pl.ds(base2 + jnp.int32(L), L)] = v0 + 1
                                c += 1
                    return c2

                lax.fori_loop(jnp.int32(0), jnp.int32(G), idx_body,
                              jnp.int32(0), unroll=False)

                pltpu.async_copy(tbl.at[idxb], rowsb, sem).wait()

                def interp_body(g, c2, lvl=lvl):
                    p = g * jnp.int32(L)
                    wx = wxb[pl.ds(p, L)]
                    wy = wyb[pl.ds(p, L)]
                    wz = wzb[pl.ds(p, L)]
                    umx = 1.0 - wx
                    umy = 1.0 - wy
                    umz = 1.0 - wz
                    p2 = p + p
                    f0 = []
                    f1 = []
                    for c in range(8):
                        base2 = jnp.int32(c * 2 * C) + p2
                        f0.append(rowsb[pl.ds(base2, L)])
                        f1.append(rowsb[pl.ds(base2 + jnp.int32(L), L)])
                    for feats, off in ((f0, 0), (f1, 1)):
                        fx00 = feats[0] * umx + feats[4] * wx
                        fx01 = feats[1] * umx + feats[5] * wx
                        fx10 = feats[2] * umx + feats[6] * wx
                        fx11 = feats[3] * umx + feats[7] * wx
                        fxy0 = fx00 * umy + fx10 * wy
                        fxy1 = fx01 * umy + fx11 * wy
                        val = fxy0 * umz + fxy1 * wz
                        outb[pl.ds(jnp.int32((2 * lvl + off) * C) + p, L)] = val
                    return c2

                lax.fori_loop(jnp.int32(0), jnp.int32(G), interp_body,
                              jnp.int32(0), unroll=False)

            for j in range(NUM_LEVELS * F):
                pltpu.sync_copy(outb.at[pl.ds(jnp.int32(j * C), C)],
                                out.at[jnp.int32(j), pl.ds(pbase, C)])
            return carry

        lax.fori_loop(jnp.int32(0), jnp.int32(NCHUNK), chunk_body,
                      jnp.int32(0), unroll=False)

    return hash_enc


def kernel(x, tables):
    shape = x.shape
    xf = x.reshape(-1, 3)
    N = xf.shape[0]
    xs = xf[:, 0]
    ys = xf[:, 1]
    zs = xf[:, 2]
    tbl = tables.reshape(NUM_LEVELS * HSIZE * F)
    out = _build(N)(xs, ys, zs, tbl)
    return out.T.reshape(*shape[:-1], NUM_LEVELS * F)


# speedup vs baseline: 1.3877x; 1.0197x over previous
import functools

import numpy as np
import jax
import jax.numpy as jnp
from jax import lax
from jax.experimental import pallas as pl
from jax.experimental.pallas import tpu as pltpu
from jax.experimental.pallas import tpu_sc as plsc

NUM_LEVELS = 16
MIN_RES = 128
MAX_RES = 2048
LOG2_HASHMAP_SIZE = 19
HSIZE = 1 << LOG2_HASHMAP_SIZE
F = 2
_b = np.exp((np.log(MAX_RES) - np.log(MIN_RES)) / (NUM_LEVELS - 1))
_RES = [int(np.floor(MIN_RES * _b ** lvl)) for lvl in range(NUM_LEVELS)]

P1 = 73856093
P2 = 19349663
P3 = 83492791

NC = 2
NS = 16
L = 16
NW = NC * NS

_RES_F = np.array(_RES, np.float32)
_RES_LO = np.repeat(_RES_F[:8], 2)
_RES_HI = np.repeat(_RES_F[8:], 2)


@functools.lru_cache(maxsize=None)
def _build(N):
    NPW = N // NW
    C = 64
    NCHUNK = NPW // C
    CI = C * 8 * 2 * L
    PW = 8 * 2 * L

    mesh = plsc.VectorSubcoreMesh(core_axis_name="c", subcore_axis_name="s")

    @functools.partial(
        pl.kernel,
        mesh=mesh,
        out_type=jax.ShapeDtypeStruct((N * NUM_LEVELS * F,), jnp.float32),
        scratch_types=[
            pltpu.VMEM((2 * L,), jnp.float32),
            pltpu.VMEM((2, C + L), jnp.float32),
            pltpu.VMEM((2, C + L), jnp.float32),
            pltpu.VMEM((2, C + L), jnp.float32),
            pltpu.VMEM((CI,), jnp.int32),
            pltpu.VMEM((CI,), jnp.int32),
            pltpu.VMEM((CI,), jnp.float32),
            pltpu.VMEM((CI,), jnp.float32),
            pltpu.VMEM((C * NUM_LEVELS * F,), jnp.float32),
            pltpu.SemaphoreType.DMA,
            pltpu.SemaphoreType.DMA,
        ],
    )
    def hash_enc(xs, ys, zs, tbl, fconst, out, fcb, xb, yb, zb, idxb0,
                 idxb1, rowsb0, rowsb1, outb, sem0, sem1):
        wid = lax.axis_index("s") * NC + lax.axis_index("c")
        lane = lax.iota(jnp.int32, L)
        pltpu.sync_copy(fconst, fcb)
        res_d = (fcb[pl.ds(jnp.int32(0), L)], fcb[pl.ds(jnp.int32(L), L)])
        resm1_d = tuple(r.astype(jnp.int32) - jnp.int32(1) for r in res_d)
        fbit = lane & jnp.int32(1)
        lvb0 = ((lane >> jnp.int32(1)) << jnp.int32(LOG2_HASHMAP_SIZE + 1)) \
            | fbit
        lvfb = (lvb0, lvb0 + jnp.int32(8 << (LOG2_HASHMAP_SIZE + 1)))
        mask = jnp.int32(HSIZE - 1)
        wbase = wid * jnp.int32(NPW)

        def fire(k, par):
            idxb = idxb0 if par == 0 else idxb1
            rowsb = rowsb0 if par == 0 else rowsb1
            sem = sem0 if par == 0 else sem1
            pbase = wbase + k * jnp.int32(C)
            pltpu.sync_copy(xs.at[pl.ds(pbase, C)], xb.at[jnp.int32(par), pl.ds(0, C)])
            pltpu.sync_copy(ys.at[pl.ds(pbase, C)], yb.at[jnp.int32(par), pl.ds(0, C)])
            pltpu.sync_copy(zs.at[pl.ds(pbase, C)], zb.at[jnp.int32(par), pl.ds(0, C)])

            def idx_body(i, c2):
                xv = jnp.broadcast_to(xb[jnp.int32(par), pl.ds(i, L)][0], (L,))
                yv = jnp.broadcast_to(yb[jnp.int32(par), pl.ds(i, L)][0], (L,))
                zv = jnp.broadcast_to(zb[jnp.int32(par), pl.ds(i, L)][0], (L,))
                base = i * jnp.int32(PW)
                for h in range(2):
                    ix = jnp.minimum((xv * res_d[h]).astype(jnp.int32),
                                     resm1_d[h])
                    iy = jnp.minimum((yv * res_d[h]).astype(jnp.int32),
                                     resm1_d[h])
                    iz = jnp.minimum((zv * res_d[h]).astype(jnp.int32),
                                     resm1_d[h])
                    hx0 = ix * jnp.int32(P1)
                    hx1 = hx0 + jnp.int32(P1)
                    hy0 = iy * jnp.int32(P2)
                    hy1 = hy0 + jnp.int32(P2)
                    hz0 = iz * jnp.int32(P3)
                    hz1 = hz0 + jnp.int32(P3)
                    c = 0
                    for hx in (hx0, hx1):
                        hxy0 = hx ^ hy0
                        hxy1 = hx ^ hy1
                        for hxy in (hxy0, hxy1):
                            for hz in (hz0, hz1):
                                v = ((((hxy ^ hz) & mask) << jnp.int32(1))
                                     | lvfb[h])
                                idxb[pl.ds(base + jnp.int32(
                                    (h * 8 + c) * L), L)] = v
                                c += 1
                return c2

            lax.fori_loop(jnp.int32(0), jnp.int32(C), idx_body,
                          jnp.int32(0), unroll=False)
            pltpu.async_copy(tbl.at[idxb], rowsb, sem)

        def drain(k, par):
            idxb = idxb0 if par == 0 else idxb1
            rowsb = rowsb0 if par == 0 else rowsb1
            sem = sem0 if par == 0 else sem1
            pbase = wbase + k * jnp.int32(C)
            pltpu.make_async_copy(tbl.at[idxb], rowsb, sem).wait()

            def interp_body(i, c2):
                xv = jnp.broadcast_to(xb[jnp.int32(par), pl.ds(i, L)][0], (L,))
                yv = jnp.broadcast_to(yb[jnp.int32(par), pl.ds(i, L)][0], (L,))
                zv = jnp.broadcast_to(zb[jnp.int32(par), pl.ds(i, L)][0], (L,))
                base = i * jnp.int32(PW)
                obase = i * jnp.int32(NUM_LEVELS * F)
                for h in range(2):
                    sx = xv * res_d[h]
                    sy = yv * res_d[h]
                    sz = zv * res_d[h]
                    wx = sx - sx.astype(jnp.int32).astype(jnp.float32)
                    wy = sy - sy.astype(jnp.int32).astype(jnp.float32)
                    wz = sz - sz.astype(jnp.int32).astype(jnp.float32)
                    umx = 1.0 - wx
                    umy = 1.0 - wy
                    umz = 1.0 - wz
                    f = [rowsb[pl.ds(base + jnp.int32((h * 8 + c) * L), L)]
                         for c in range(8)]
                    fx00 = f[0] * umx + f[4] * wx
                    fx01 = f[1] * umx + f[5] * wx
                    fx10 = f[2] * umx + f[6] * wx
                    fx11 = f[3] * umx + f[7] * wx
                    fxy0 = fx00 * umy + fx10 * wy
                    fxy1 = fx01 * umy + fx11 * wy
                    val = fxy0 * umz + fxy1 * wz
                    outb[pl.ds(obase + jnp.int32(h * L), L)] = val
                return c2

            lax.fori_loop(jnp.int32(0), jnp.int32(C), interp_body,
                          jnp.int32(0), unroll=False)
            pltpu.sync_copy(
                outb,
                out.at[pl.ds(pbase * jnp.int32(NUM_LEVELS * F),
                             C * NUM_LEVELS * F)])

        fire(jnp.int32(0), 0)

        def chunk_body(j, carry):
            k = j * jnp.int32(2)
            fire(k + jnp.int32(1), 1)
            drain(k, 0)
            fire(k + jnp.int32(2), 0)
            drain(k + jnp.int32(1), 1)
            return carry

        lax.fori_loop(jnp.int32(0), jnp.int32(NCHUNK // 2 - 1), chunk_body,
                      jnp.int32(0), unroll=False)
        klast = jnp.int32(NCHUNK - 2)
        fire(klast + jnp.int32(1), 1)
        drain(klast, 0)
        drain(klast + jnp.int32(1), 1)

    return hash_enc


def kernel(x, tables):
    shape = x.shape
    xf = x.reshape(-1, 3)
    N = xf.shape[0]
    tbl = tables.reshape(NUM_LEVELS * HSIZE * F)
    fconst = jnp.asarray(np.concatenate([_RES_LO, _RES_HI]))
    out = _build(N)(xf[:, 0], xf[:, 1], xf[:, 2], tbl, fconst)
    return out.reshape(*shape[:-1], NUM_LEVELS * F)


# speedup vs baseline: 7.1180x; 5.1294x over previous
import functools

import numpy as np
import jax
import jax.numpy as jnp
from jax import lax
from jax.experimental import pallas as pl
from jax.experimental.pallas import tpu as pltpu
from jax.experimental.pallas import tpu_sc as plsc

NUM_LEVELS = 16
MIN_RES = 128
MAX_RES = 2048
LOG2_HASHMAP_SIZE = 19
HSIZE = 1 << LOG2_HASHMAP_SIZE
F = 2
_b = np.exp((np.log(MAX_RES) - np.log(MIN_RES)) / (NUM_LEVELS - 1))
_RES = [int(np.floor(MIN_RES * _b ** lvl)) for lvl in range(NUM_LEVELS)]

P1 = 73856093
P2 = 19349663
P3 = 83492791

NC = 2
NS = 16
L = 16
NW = NC * NS

_RES_F = np.array(_RES, np.float32)
_RES_LO = np.repeat(_RES_F[:8], 2)
_RES_HI = np.repeat(_RES_F[8:], 2)


@functools.lru_cache(maxsize=None)
def _build(N):
    NPW = N // NW
    C = 64
    NCHUNK = NPW // C
    CI = C * 8 * 2 * L
    PW = 8 * 2 * L

    mesh = plsc.VectorSubcoreMesh(core_axis_name="c", subcore_axis_name="s")

    @functools.partial(
        pl.kernel,
        mesh=mesh,
        out_type=jax.ShapeDtypeStruct((N * NUM_LEVELS * F,), jnp.float32),
        scratch_types=[
            pltpu.VMEM((2 * L,), jnp.float32),
            pltpu.VMEM((2, C + L), jnp.float32),
            pltpu.VMEM((2, C + L), jnp.float32),
            pltpu.VMEM((2, C + L), jnp.float32),
            pltpu.VMEM((CI,), jnp.int32),
            pltpu.VMEM((CI,), jnp.int32),
            pltpu.VMEM((CI,), jnp.float32),
            pltpu.VMEM((CI,), jnp.float32),
            pltpu.VMEM((C * NUM_LEVELS * F,), jnp.float32),
            pltpu.SemaphoreType.DMA,
            pltpu.SemaphoreType.DMA,
        ],
    )
    def hash_enc(xs, ys, zs, tbl, fconst, out, fcb, xb, yb, zb, idxb0,
                 idxb1, rowsb0, rowsb1, outb, sem0, sem1):
        wid = lax.axis_index("s") * NC + lax.axis_index("c")
        lane = lax.iota(jnp.int32, L)
        pltpu.sync_copy(fconst, fcb)
        res_d = (fcb[pl.ds(jnp.int32(0), L)], fcb[pl.ds(jnp.int32(L), L)])
        resm1_d = tuple(r.astype(jnp.int32) - jnp.int32(1) for r in res_d)
        fbit = lane & jnp.int32(1)
        lvb0 = ((lane >> jnp.int32(1)) << jnp.int32(LOG2_HASHMAP_SIZE + 1)) \
            | (fbit << jnp.int32(7))
        lvfb = (lvb0, lvb0 + jnp.int32(8 << (LOG2_HASHMAP_SIZE + 1)))
        mask = jnp.int32(HSIZE - 1)
        low7 = jnp.int32(127)
        wbase = wid * jnp.int32(NPW)

        def fire(k, par):
            idxb = idxb0 if par == 0 else idxb1
            rowsb = rowsb0 if par == 0 else rowsb1
            sem = sem0 if par == 0 else sem1
            pbase = wbase + k * jnp.int32(C)
            pltpu.sync_copy(xs.at[pl.ds(pbase, C)], xb.at[jnp.int32(par), pl.ds(0, C)])
            pltpu.sync_copy(ys.at[pl.ds(pbase, C)], yb.at[jnp.int32(par), pl.ds(0, C)])
            pltpu.sync_copy(zs.at[pl.ds(pbase, C)], zb.at[jnp.int32(par), pl.ds(0, C)])

            def idx_body(i, c2):
                xv = jnp.broadcast_to(xb[jnp.int32(par), pl.ds(i, L)][0], (L,))
                yv = jnp.broadcast_to(yb[jnp.int32(par), pl.ds(i, L)][0], (L,))
                zv = jnp.broadcast_to(zb[jnp.int32(par), pl.ds(i, L)][0], (L,))
                base = i * jnp.int32(PW)
                for h in range(2):
                    ix = jnp.minimum((xv * res_d[h]).astype(jnp.int32),
                                     resm1_d[h])
                    iy = jnp.minimum((yv * res_d[h]).astype(jnp.int32),
                                     resm1_d[h])
                    iz = jnp.minimum((zv * res_d[h]).astype(jnp.int32),
                                     resm1_d[h])
                    hx0 = ix * jnp.int32(P1)
                    hx1 = hx0 + jnp.int32(P1)
                    hy0 = iy * jnp.int32(P2)
                    hy1 = hy0 + jnp.int32(P2)
                    hz0 = iz * jnp.int32(P3)
                    hz1 = hz0 + jnp.int32(P3)
                    c = 0
                    for hx in (hx0, hx1):
                        hxy0 = hx ^ hy0
                        hxy1 = hx ^ hy1
                        for hxy in (hxy0, hxy1):
                            for hz in (hz0, hz1):
                                t = (hxy ^ hz) & mask
                                v = (((t >> jnp.int32(7)) << jnp.int32(8))
                                     | (t & low7) | lvfb[h])
                                idxb[pl.ds(base + jnp.int32(
                                    (h * 8 + c) * L), L)] = v
                                c += 1
                return c2

            lax.fori_loop(jnp.int32(0), jnp.int32(C), idx_body,
                          jnp.int32(0), unroll=False)
            pltpu.async_copy(tbl.at[idxb], rowsb, sem)

        def drain(k, par):
            idxb = idxb0 if par == 0 else idxb1
            rowsb = rowsb0 if par == 0 else rowsb1
            sem = sem0 if par == 0 else sem1
            pbase = wbase + k * jnp.int32(C)
            pltpu.make_async_copy(tbl.at[idxb], rowsb, sem).wait()

            def interp_body(i, c2):
                xv = jnp.broadcast_to(xb[jnp.int32(par), pl.ds(i, L)][0], (L,))
                yv = jnp.broadcast_to(yb[jnp.int32(par), pl.ds(i, L)][0], (L,))
                zv = jnp.broadcast_to(zb[jnp.int32(par), pl.ds(i, L)][0], (L,))
                base = i * jnp.int32(PW)
                obase = i * jnp.int32(NUM_LEVELS * F)
                for h in range(2):
                    sx = xv * res_d[h]
                    sy = yv * res_d[h]
                    sz = zv * res_d[h]
                    wx = sx - sx.astype(jnp.int32).astype(jnp.float32)
                    wy = sy - sy.astype(jnp.int32).astype(jnp.float32)
                    wz = sz - sz.astype(jnp.int32).astype(jnp.float32)
                    umx = 1.0 - wx
                    umy = 1.0 - wy
                    umz = 1.0 - wz
                    f = [rowsb[pl.ds(base + jnp.int32((h * 8 + c) * L), L)]
                         for c in range(8)]
                    fx00 = f[0] * umx + f[4] * wx
                    fx01 = f[1] * umx + f[5] * wx
                    fx10 = f[2] * umx + f[6] * wx
                    fx11 = f[3] * umx + f[7] * wx
                    fxy0 = fx00 * umy + fx10 * wy
                    fxy1 = fx01 * umy + fx11 * wy
                    val = fxy0 * umz + fxy1 * wz
                    outb[pl.ds(obase + jnp.int32(h * L), L)] = val
                return c2

            lax.fori_loop(jnp.int32(0), jnp.int32(C), interp_body,
                          jnp.int32(0), unroll=False)
            pltpu.sync_copy(
                outb,
                out.at[pl.ds(pbase * jnp.int32(NUM_LEVELS * F),
                             C * NUM_LEVELS * F)])

        fire(jnp.int32(0), 0)

        def chunk_body(j, carry):
            k = j * jnp.int32(2)
            fire(k + jnp.int32(1), 1)
            drain(k, 0)
            fire(k + jnp.int32(2), 0)
            drain(k + jnp.int32(1), 1)
            return carry

        lax.fori_loop(jnp.int32(0), jnp.int32(NCHUNK // 2 - 1), chunk_body,
                      jnp.int32(0), unroll=False)
        klast = jnp.int32(NCHUNK - 2)
        fire(klast + jnp.int32(1), 1)
        drain(klast, 0)
        drain(klast + jnp.int32(1), 1)

    return hash_enc


def kernel(x, tables):
    shape = x.shape
    xf = x.reshape(-1, 3)
    N = xf.shape[0]
    tbl = tables.reshape(NUM_LEVELS, HSIZE // 128, 128, F) \
        .transpose(0, 1, 3, 2).reshape(-1)
    fconst = jnp.asarray(np.concatenate([_RES_LO, _RES_HI]))
    out = _build(N)(xf[:, 0], xf[:, 1], xf[:, 2], tbl, fconst)
    return out.reshape(*shape[:-1], NUM_LEVELS * F)
